# baseline (speedup 1.0000x reference)
import jax
import jax.numpy as jnp
from jax import lax
from jax.experimental import pallas as pl
from jax.experimental.pallas import tpu as pltpu
from jax.experimental.pallas import tpu_sc as plsc

N = 2048
C = 1024
E = 8
F = 4096
CAP = 768
EP = 128
NS = 16
NC = 2
NW = NC * NS
ROWS = E * CAP
RPT = ROWS // NW
CHUNK = 64
NCHUNK = RPT // CHUNK
GC = 48
NGC = RPT // GC
TPT = N // NW
FB = 1024
NFB = F // FB


def _router_body(lg_ref, i1_ref, i2_ref, p1_ref, p2_ref):
    lg = lg_ref[...]
    iot = lax.broadcasted_iota(jnp.int32, (N, EP), 1)
    v1 = jnp.max(lg, axis=1, keepdims=True)
    i1 = jnp.min(jnp.where(lg >= v1, iot, EP), axis=1, keepdims=True)
    lg2 = jnp.where(iot == i1, -1e30, lg)
    v2 = jnp.max(lg2, axis=1, keepdims=True)
    i2 = jnp.min(jnp.where(lg2 >= v2, iot, EP), axis=1, keepdims=True)
    p1 = 1.0 / (1.0 + jnp.exp(v2 - v1))
    i1_ref[...] = i1
    i2_ref[...] = i2
    p1_ref[...] = p1
    p2_ref[...] = 1.0 - p1


def _router(lg_p):
    return pl.pallas_call(
        _router_body,
        out_shape=[
            jax.ShapeDtypeStruct((N, 1), jnp.int32),
            jax.ShapeDtypeStruct((N, 1), jnp.int32),
            jax.ShapeDtypeStruct((N, 1), jnp.float32),
            jax.ShapeDtypeStruct((N, 1), jnp.float32),
        ],
    )(lg_p)


def _scan_body(i1_hbm, i2_hbm, p1_hbm, p2_hbm,
               tok_hbm, w_hbm, g0p_hbm, g1p_hbm, cnt_hbm,
               i1_v, i2_v, p1_v, p2_v, tok_v, w_v, g0_v, g1_v, cnt_v):
    c = lax.axis_index("c")
    s = lax.axis_index("s")
    wid = s * NC + c

    @pl.when(wid < E)
    def _():
        e = wid
        pltpu.sync_copy(i1_hbm, i1_v)
        pltpu.sync_copy(i2_hbm, i2_v)
        pltpu.sync_copy(p1_hbm, p1_v)
        pltpu.sync_copy(p2_hbm, p2_v)

        z16i = jnp.zeros((16,), jnp.int32)
        z16f = jnp.zeros((16,), jnp.float32)

        def _zero_cap(i, _):
            tok_v[pl.ds(i * 16, 16)] = z16i
            w_v[pl.ds(i * 16, 16)] = z16f
            return 0
        lax.fori_loop(0, CAP // 16, _zero_cap, 0)

        def _zero_g(i, _):
            g0_v[pl.ds(i * 16, 16)] = z16i
            g1_v[pl.ds(i * 16, 16)] = z16i
            return 0
        lax.fori_loop(0, N // 16, _zero_g, 0)

        evec = z16i + e
        ebase = e * CAP

        def _scan(j, carry):
            a1 = i1_v[pl.ds(j * 16, 16)]
            a2 = i2_v[pl.ds(j * 16, 16)]
            m0 = a1 == evec
            m1 = a2 == evec
            o0 = jnp.where(m0, 1, 0).astype(jnp.int32)
            o1 = jnp.where(m1, 1, 0).astype(jnp.int32)
            c0 = plsc.cumsum(o0)
            c1 = plsc.cumsum(o1)
            prior = (c0 - o0) + (c1 - o1)
            r0 = carry + prior
            r1 = r0 + o0
            toks = lax.iota(jnp.int32, 16) + j * 16
            k0 = m0 & (r0 < CAP)
            k1 = m1 & (r1 < CAP)
            plsc.store_scatter(tok_v, [r0], toks, mask=k0)
            plsc.store_scatter(tok_v, [r1], toks, mask=k1)
            plsc.store_scatter(w_v, [r0], p1_v[pl.ds(j * 16, 16)], mask=k0)
            plsc.store_scatter(w_v, [r1], p2_v[pl.ds(j * 16, 16)], mask=k1)
            enc0 = jnp.where(k0, ebase + r0 + 2, 1)
            enc1 = jnp.where(k1, ebase + r1 + 2, 1)
            plsc.store_scatter(g0_v, [toks], enc0, mask=m0)
            plsc.store_scatter(g1_v, [toks], enc1, mask=m1)
            return (carry
                    + plsc.all_reduce_population_count(m0)
                    + plsc.all_reduce_population_count(m1))

        total = lax.fori_loop(0, N // 16, _scan, jnp.zeros((16,), jnp.int32))
        cnt_v[...] = total

        pltpu.sync_copy(tok_v, tok_hbm.at[e])
        pltpu.sync_copy(w_v, w_hbm.at[e])
        pltpu.sync_copy(g0_v, g0p_hbm.at[e])
        pltpu.sync_copy(g1_v, g1p_hbm.at[e])
        pltpu.sync_copy(cnt_v, cnt_hbm.at[e])


def _scan(i1f, i2f, p1f, p2f):
    mesh = plsc.VectorSubcoreMesh(core_axis_name="c", subcore_axis_name="s")
    return pl.kernel(
        _scan_body,
        out_type=[
            jax.ShapeDtypeStruct((E, CAP), jnp.int32),
            jax.ShapeDtypeStruct((E, CAP), jnp.float32),
            jax.ShapeDtypeStruct((E, N), jnp.int32),
            jax.ShapeDtypeStruct((E, N), jnp.int32),
            jax.ShapeDtypeStruct((E, 16), jnp.int32),
        ],
        mesh=mesh,
        compiler_params=pltpu.CompilerParams(needs_layout_passes=False),
        scratch_types=[
            pltpu.VMEM((N,), jnp.int32),
            pltpu.VMEM((N,), jnp.int32),
            pltpu.VMEM((N,), jnp.float32),
            pltpu.VMEM((N,), jnp.float32),
            pltpu.VMEM((CAP,), jnp.int32),
            pltpu.VMEM((CAP,), jnp.float32),
            pltpu.VMEM((N,), jnp.int32),
            pltpu.VMEM((N,), jnp.int32),
            pltpu.VMEM((16,), jnp.int32),
        ],
    )(i1f, i2f, p1f, p2f)


def _merge_body(g0pf_hbm, g1pf_hbm, cntf_hbm, tokf_hbm, x_hbm,
                xd_hbm, g0_hbm, g1_hbm,
                gbuf_v, cnt_v, enc0_v, enc1_v, idx_v, rowsa_v, rowsb_v,
                gsem, wsem):
    c = lax.axis_index("c")
    s = lax.axis_index("s")
    wid = s * NC + c
    tbase = wid * TPT

    pltpu.sync_copy(tokf_hbm.at[pl.ds(wid * RPT, RPT)], idx_v)
    bufs = [rowsa_v, rowsb_v]
    gh = [None] * NGC
    wh = [None] * NGC
    gh[0] = pltpu.async_copy(x_hbm.at[idx_v.at[pl.ds(0, GC)]], bufs[0], gsem)

    pltpu.sync_copy(g0pf_hbm, gbuf_v)
    for i in range(TPT // 16):
        d = pl.ds(i * 16, 16)
        acc = gbuf_v[pl.ds(tbase + i * 16, 16)]
        for e in range(1, E):
            acc = acc + gbuf_v[pl.ds(e * N + tbase + i * 16, 16)]
        enc0_v[d] = acc
    pltpu.sync_copy(g1pf_hbm, gbuf_v)
    for i in range(TPT // 16):
        d = pl.ds(i * 16, 16)
        acc = gbuf_v[pl.ds(tbase + i * 16, 16)]
        for e in range(1, E):
            acc = acc + gbuf_v[pl.ds(e * N + tbase + i * 16, 16)]
        enc1_v[d] = acc

    pltpu.sync_copy(cntf_hbm, cnt_v)
    best = jnp.zeros((16,), jnp.int32) + (1 << 30)
    for e in range(E):
        best = jnp.minimum(best, cnt_v[pl.ds(e * 16, 16)] * 8 + e)
    zero_row = (best & 7) * CAP + (best >> 3)

    for i in range(TPT // 16):
        d = pl.ds(i * 16, 16)
        v0 = enc0_v[d]
        v1 = enc1_v[d]
        enc0_v[d] = jnp.where(v0 >= 2, v0 - 2, zero_row)
        enc1_v[d] = jnp.where(v1 >= 2, v1 - 2, zero_row)
    pltpu.sync_copy(enc0_v, g0_hbm.at[pl.ds(tbase, TPT)])
    pltpu.sync_copy(enc1_v, g1_hbm.at[pl.ds(tbase, TPT)])

    for k in range(NGC):
        gh[k].wait()
        if k >= 2:
            wh[k - 2].wait()
        wh[k] = pltpu.async_copy(
            bufs[k % 2], xd_hbm.at[pl.ds(wid * RPT + k * GC, GC)], wsem)
        if k + 1 < NGC:
            gh[k + 1] = pltpu.async_copy(
                x_hbm.at[idx_v.at[pl.ds((k + 1) * GC, GC)]],
                bufs[(k + 1) % 2], gsem)
    wh[NGC - 2].wait()
    wh[NGC - 1].wait()


def _merge(g0pf, g1pf, cntf, tokf, xf):
    mesh = plsc.VectorSubcoreMesh(core_axis_name="c", subcore_axis_name="s")
    return pl.kernel(
        _merge_body,
        out_type=[
            jax.ShapeDtypeStruct((ROWS, C // 2), jnp.float32),
            jax.ShapeDtypeStruct((N,), jnp.int32),
            jax.ShapeDtypeStruct((N,), jnp.int32),
        ],
        mesh=mesh,
        compiler_params=pltpu.CompilerParams(needs_layout_passes=False),
        scratch_types=[
            pltpu.VMEM((E * N,), jnp.int32),
            pltpu.VMEM((E * 16,), jnp.int32),
            pltpu.VMEM((TPT,), jnp.int32),
            pltpu.VMEM((TPT,), jnp.int32),
            pltpu.VMEM((RPT,), jnp.int32),
            pltpu.VMEM((GC, C // 2), jnp.float32),
            pltpu.VMEM((GC, C // 2), jnp.float32),
            pltpu.SemaphoreType.DMA,
            pltpu.SemaphoreType.DMA,
        ],
    )(g0pf, g1pf, cntf, tokf, xf)


def _ffn_body(xd_ref, w1_ref, b1_ref, w2_ref, b2_ref, wc_ref, yd_ref, acc_ref):
    fb = pl.program_id(1)
    h = jnp.dot(xd_ref[...],
                w1_ref[0].astype(jnp.bfloat16),
                preferred_element_type=jnp.float32) + b1_ref[0]
    h = jnp.maximum(h, 0.0)
    part = jnp.dot(h.astype(jnp.bfloat16), w2_ref[0].astype(jnp.bfloat16),
                   preferred_element_type=jnp.float32)

    @pl.when(fb == 0)
    def _():
        acc_ref[...] = part

    @pl.when(fb > 0)
    def _():
        acc_ref[...] += part

    @pl.when(fb == NFB - 1)
    def _():
        yd_ref[...] = (acc_ref[...] + b2_ref[0]) * wc_ref[0]


def _ffn(xd, W1, b1, W2, b2, wc):
    return pl.pallas_call(
        _ffn_body,
        grid=(E, NFB),
        in_specs=[
            pl.BlockSpec((CAP, C), lambda e, f: (e, 0)),
            pl.BlockSpec((1, C, FB), lambda e, f: (e, 0, f)),
            pl.BlockSpec((1, 1, FB), lambda e, f: (e, 0, f)),
            pl.BlockSpec((1, FB, C), lambda e, f: (e, f, 0)),
            pl.BlockSpec((1, 1, C), lambda e, f: (e, 0, 0)),
            pl.BlockSpec((1, CAP, 1), lambda e, f: (e, 0, 0)),
        ],
        out_specs=pl.BlockSpec((CAP, C), lambda e, f: (e, 0)),
        out_shape=jax.ShapeDtypeStruct((ROWS, C), jnp.float32),
        scratch_shapes=[pltpu.VMEM((CAP, C), jnp.float32)],
        compiler_params=pltpu.CompilerParams(
            dimension_semantics=("arbitrary", "arbitrary")),
    )(xd, W1, b1, W2, b2, wc)


def _combine_body(yd_hbm, g0_hbm, g1_hbm,
                  out_hbm,
                  g0_v, g1_v, bufa_v, bufb_v, bufc_v, gsem, wsem):
    c = lax.axis_index("c")
    s = lax.axis_index("s")
    wid = s * NC + c
    tbase = wid * TPT
    half = TPT // 2

    pltpu.sync_copy(g0_hbm.at[pl.ds(tbase, TPT)], g0_v)
    pltpu.sync_copy(g1_hbm.at[pl.ds(tbase, TPT)], g1_v)

    ga = pltpu.async_copy(yd_hbm.at[g0_v.at[pl.ds(0, half)]], bufa_v, gsem)
    gb = pltpu.async_copy(yd_hbm.at[g1_v.at[pl.ds(0, half)]], bufb_v, gsem)
    ga.wait()
    gb.wait()

    @plsc.parallel_loop(0, half * C // 16, 1, unroll=8)
    def _add0(i):
        r = i >> 6
        d = pl.ds((i & 63) * 16, 16)
        bufa_v[r, d] = bufa_v[r, d] + bufb_v[r, d]

    gc2 = pltpu.async_copy(yd_hbm.at[g0_v.at[pl.ds(half, half)]], bufc_v, gsem)
    gb2 = pltpu.async_copy(yd_hbm.at[g1_v.at[pl.ds(half, half)]], bufb_v, gsem)
    wa = pltpu.async_copy(bufa_v, out_hbm.at[pl.ds(tbase, half)], wsem)
    gc2.wait()
    gb2.wait()

    @plsc.parallel_loop(0, half * C // 16, 1, unroll=8)
    def _add1(i):
        r = i >> 6
        d = pl.ds((i & 63) * 16, 16)
        bufc_v[r, d] = bufc_v[r, d] + bufb_v[r, d]

    wa.wait()
    pltpu.sync_copy(bufc_v, out_hbm.at[pl.ds(tbase + half, half)])


def _combine(yd, g0, g1):
    mesh = plsc.VectorSubcoreMesh(core_axis_name="c", subcore_axis_name="s")
    return pl.kernel(
        _combine_body,
        out_type=jax.ShapeDtypeStruct((N, C), jnp.float32),
        mesh=mesh,
        compiler_params=pltpu.CompilerParams(needs_layout_passes=False),
        scratch_types=[
            pltpu.VMEM((TPT,), jnp.int32),
            pltpu.VMEM((TPT,), jnp.int32),
            pltpu.VMEM((TPT // 2, C), jnp.float32),
            pltpu.VMEM((TPT // 2, C), jnp.float32),
            pltpu.VMEM((TPT // 2, C), jnp.float32),
            pltpu.SemaphoreType.DMA,
            pltpu.SemaphoreType.DMA,
        ],
    )(yd, g0, g1)


def kernel(x, Wr, br, W1, b1, W2, b2):
    xf = x.reshape(N, C)
    lgs = xf @ Wr + br
    lg_p = jnp.pad(lgs, ((0, 0), (0, EP - E)), constant_values=-1e30)

    i1, i2, p1, p2 = _router(lg_p)
    tok, w, g0p, g1p, cnt = _scan(i1.reshape(N), i2.reshape(N),
                                  p1.reshape(N), p2.reshape(N))
    xb2 = lax.bitcast_convert_type(
        xf.astype(jnp.bfloat16).reshape(N, C // 2, 2), jnp.float32)
    xd2, g0, g1 = _merge(g0p.reshape(E * N), g1p.reshape(E * N),
                         cnt.reshape(E * 16), tok.reshape(ROWS), xb2)
    xd = lax.bitcast_convert_type(xd2, jnp.bfloat16).reshape(ROWS, C)
    yd = _ffn(xd, W1, b1.reshape(E, 1, F), W2, b2.reshape(E, 1, C),
              w.reshape(E, CAP, 1))
    out = _combine(yd, g0, g1)
    return out.reshape(1, N, C)


# speedup vs baseline: 1.9529x; 1.9529x over previous
import jax
import jax.numpy as jnp
from jax import lax
from jax.experimental import pallas as pl
from jax.experimental.pallas import tpu as pltpu
from jax.experimental.pallas import tpu_sc as plsc

N = 2048
C = 1024
E = 8
F = 4096
CAP = 768
EP = 128
NS = 16
NC = 2
NW = NC * NS
ROWS = E * CAP
RPT = ROWS // NW
CHUNK = 64
NCHUNK = RPT // CHUNK
GC = 48
NGC = RPT // GC
TPT = N // NW
FB = 1024
NFB = F // FB


def _router_body(lg_ref, i1_ref, i2_ref, p1_ref, p2_ref):
    lg = lg_ref[...]
    iot = lax.broadcasted_iota(jnp.int32, (N, EP), 1)
    v1 = jnp.max(lg, axis=1, keepdims=True)
    i1 = jnp.min(jnp.where(lg >= v1, iot, EP), axis=1, keepdims=True)
    lg2 = jnp.where(iot == i1, -1e30, lg)
    v2 = jnp.max(lg2, axis=1, keepdims=True)
    i2 = jnp.min(jnp.where(lg2 >= v2, iot, EP), axis=1, keepdims=True)
    p1 = 1.0 / (1.0 + jnp.exp(v2 - v1))
    i1_ref[...] = i1
    i2_ref[...] = i2
    p1_ref[...] = p1
    p2_ref[...] = 1.0 - p1


def _router(lg_p):
    return pl.pallas_call(
        _router_body,
        out_shape=[
            jax.ShapeDtypeStruct((N, 1), jnp.int32),
            jax.ShapeDtypeStruct((N, 1), jnp.int32),
            jax.ShapeDtypeStruct((N, 1), jnp.float32),
            jax.ShapeDtypeStruct((N, 1), jnp.float32),
        ],
    )(lg_p)


def _scan_body(i1_hbm, i2_hbm, p1_hbm, p2_hbm,
               tok_hbm, w_hbm, g0p_hbm, g1p_hbm, cnt_hbm,
               i1_v, i2_v, p1_v, p2_v, tok_v, w_v, g0_v, g1_v, cnt_v):
    c = lax.axis_index("c")
    s = lax.axis_index("s")
    wid = s * NC + c

    @pl.when(wid < E)
    def _():
        e = wid
        pltpu.sync_copy(i1_hbm, i1_v)
        pltpu.sync_copy(i2_hbm, i2_v)
        pltpu.sync_copy(p1_hbm, p1_v)
        pltpu.sync_copy(p2_hbm, p2_v)

        z16i = jnp.zeros((16,), jnp.int32)
        z16f = jnp.zeros((16,), jnp.float32)

        def _zero_cap(i, _):
            tok_v[pl.ds(i * 16, 16)] = z16i
            w_v[pl.ds(i * 16, 16)] = z16f
            return 0
        lax.fori_loop(0, CAP // 16, _zero_cap, 0)

        def _zero_g(i, _):
            g0_v[pl.ds(i * 16, 16)] = z16i
            g1_v[pl.ds(i * 16, 16)] = z16i
            return 0
        lax.fori_loop(0, N // 16, _zero_g, 0)

        evec = z16i + e
        ebase = e * CAP

        def _scan(j, carry):
            a1 = i1_v[pl.ds(j * 16, 16)]
            a2 = i2_v[pl.ds(j * 16, 16)]
            m0 = a1 == evec
            m1 = a2 == evec
            o0 = jnp.where(m0, 1, 0).astype(jnp.int32)
            o1 = jnp.where(m1, 1, 0).astype(jnp.int32)
            c0 = plsc.cumsum(o0)
            c1 = plsc.cumsum(o1)
            prior = (c0 - o0) + (c1 - o1)
            r0 = carry + prior
            r1 = r0 + o0
            toks = lax.iota(jnp.int32, 16) + j * 16
            k0 = m0 & (r0 < CAP)
            k1 = m1 & (r1 < CAP)
            plsc.store_scatter(tok_v, [r0], toks, mask=k0)
            plsc.store_scatter(tok_v, [r1], toks, mask=k1)
            plsc.store_scatter(w_v, [r0], p1_v[pl.ds(j * 16, 16)], mask=k0)
            plsc.store_scatter(w_v, [r1], p2_v[pl.ds(j * 16, 16)], mask=k1)
            enc0 = jnp.where(k0, ebase + r0 + 2, 1)
            enc1 = jnp.where(k1, ebase + r1 + 2, 1)
            plsc.store_scatter(g0_v, [toks], enc0, mask=m0)
            plsc.store_scatter(g1_v, [toks], enc1, mask=m1)
            return (carry
                    + plsc.all_reduce_population_count(m0)
                    + plsc.all_reduce_population_count(m1))

        total = lax.fori_loop(0, N // 16, _scan, jnp.zeros((16,), jnp.int32))
        cnt_v[...] = total

        pltpu.sync_copy(tok_v, tok_hbm.at[e])
        pltpu.sync_copy(w_v, w_hbm.at[e])
        pltpu.sync_copy(g0_v, g0p_hbm.at[e])
        pltpu.sync_copy(g1_v, g1p_hbm.at[e])
        pltpu.sync_copy(cnt_v, cnt_hbm.at[e])


def _scan(i1f, i2f, p1f, p2f):
    mesh = plsc.VectorSubcoreMesh(core_axis_name="c", subcore_axis_name="s")
    return pl.kernel(
        _scan_body,
        out_type=[
            jax.ShapeDtypeStruct((E, CAP), jnp.int32),
            jax.ShapeDtypeStruct((E, CAP), jnp.float32),
            jax.ShapeDtypeStruct((E, N), jnp.int32),
            jax.ShapeDtypeStruct((E, N), jnp.int32),
            jax.ShapeDtypeStruct((E, 16), jnp.int32),
        ],
        mesh=mesh,
        compiler_params=pltpu.CompilerParams(needs_layout_passes=False),
        scratch_types=[
            pltpu.VMEM((N,), jnp.int32),
            pltpu.VMEM((N,), jnp.int32),
            pltpu.VMEM((N,), jnp.float32),
            pltpu.VMEM((N,), jnp.float32),
            pltpu.VMEM((CAP,), jnp.int32),
            pltpu.VMEM((CAP,), jnp.float32),
            pltpu.VMEM((N,), jnp.int32),
            pltpu.VMEM((N,), jnp.int32),
            pltpu.VMEM((16,), jnp.int32),
        ],
    )(i1f, i2f, p1f, p2f)


def _merge_body(g0pf_hbm, g1pf_hbm, cntf_hbm,
                g0_hbm, g1_hbm,
                gbuf_v, cnt_v, enc0_v, enc1_v):
    c = lax.axis_index("c")
    s = lax.axis_index("s")
    wid = s * NC + c
    tbase = wid * TPT

    pltpu.sync_copy(g0pf_hbm, gbuf_v)
    for i in range(TPT // 16):
        d = pl.ds(i * 16, 16)
        acc = gbuf_v[pl.ds(tbase + i * 16, 16)]
        for e in range(1, E):
            acc = acc + gbuf_v[pl.ds(e * N + tbase + i * 16, 16)]
        enc0_v[d] = acc
    pltpu.sync_copy(g1pf_hbm, gbuf_v)
    for i in range(TPT // 16):
        d = pl.ds(i * 16, 16)
        acc = gbuf_v[pl.ds(tbase + i * 16, 16)]
        for e in range(1, E):
            acc = acc + gbuf_v[pl.ds(e * N + tbase + i * 16, 16)]
        enc1_v[d] = acc

    pltpu.sync_copy(cntf_hbm, cnt_v)
    best = jnp.zeros((16,), jnp.int32) + (1 << 30)
    for e in range(E):
        best = jnp.minimum(best, cnt_v[pl.ds(e * 16, 16)] * 8 + e)
    zero_row = (best & 7) * CAP + (best >> 3)

    for i in range(TPT // 16):
        d = pl.ds(i * 16, 16)
        v0 = enc0_v[d]
        v1 = enc1_v[d]
        enc0_v[d] = jnp.where(v0 >= 2, v0 - 2, zero_row)
        enc1_v[d] = jnp.where(v1 >= 2, v1 - 2, zero_row)
    pltpu.sync_copy(enc0_v, g0_hbm.at[pl.ds(tbase, TPT)])
    pltpu.sync_copy(enc1_v, g1_hbm.at[pl.ds(tbase, TPT)])


def _merge(g0pf, g1pf, cntf):
    mesh = plsc.VectorSubcoreMesh(core_axis_name="c", subcore_axis_name="s")
    return pl.kernel(
        _merge_body,
        out_type=[
            jax.ShapeDtypeStruct((N,), jnp.int32),
            jax.ShapeDtypeStruct((N,), jnp.int32),
        ],
        mesh=mesh,
        compiler_params=pltpu.CompilerParams(needs_layout_passes=False),
        scratch_types=[
            pltpu.VMEM((E * N,), jnp.int32),
            pltpu.VMEM((E * 16,), jnp.int32),
            pltpu.VMEM((TPT,), jnp.int32),
            pltpu.VMEM((TPT,), jnp.int32),
        ],
    )(g0pf, g1pf, cntf)


def _ffn_body(tok_ref, xb_ref, w1_ref, b1_ref, w2_ref, b2_ref, wc_ref,
              yd_ref, xg_ref, acc_ref):
    fb = pl.program_id(1)

    @pl.when(fb == 0)
    def _():
        iot = lax.broadcasted_iota(jnp.int32, (CAP, N), 1)
        p = (iot == tok_ref[0]).astype(jnp.bfloat16)
        xg_ref[...] = jnp.dot(p, xb_ref[...],
                              preferred_element_type=jnp.float32
                              ).astype(jnp.bfloat16)

    h = jnp.dot(xg_ref[...], w1_ref[0].astype(jnp.bfloat16),
                preferred_element_type=jnp.float32) + b1_ref[0]
    h = jnp.maximum(h, 0.0)
    part = jnp.dot(h.astype(jnp.bfloat16), w2_ref[0].astype(jnp.bfloat16),
                   preferred_element_type=jnp.float32)

    @pl.when(fb == 0)
    def _():
        acc_ref[...] = part

    @pl.when(fb > 0)
    def _():
        acc_ref[...] += part

    @pl.when(fb == NFB - 1)
    def _():
        yd_ref[...] = (acc_ref[...] + b2_ref[0]) * wc_ref[0]


def _ffn(tok3, xb, W1, b1, W2, b2, wc):
    return pl.pallas_call(
        _ffn_body,
        grid=(E, NFB),
        in_specs=[
            pl.BlockSpec((1, CAP, 1), lambda e, f: (e, 0, 0)),
            pl.BlockSpec((N, C), lambda e, f: (0, 0)),
            pl.BlockSpec((1, C, FB), lambda e, f: (e, 0, f)),
            pl.BlockSpec((1, 1, FB), lambda e, f: (e, 0, f)),
            pl.BlockSpec((1, FB, C), lambda e, f: (e, f, 0)),
            pl.BlockSpec((1, 1, C), lambda e, f: (e, 0, 0)),
            pl.BlockSpec((1, CAP, 1), lambda e, f: (e, 0, 0)),
        ],
        out_specs=pl.BlockSpec((CAP, C), lambda e, f: (e, 0)),
        out_shape=jax.ShapeDtypeStruct((ROWS, C), jnp.float32),
        scratch_shapes=[pltpu.VMEM((CAP, C), jnp.bfloat16),
                        pltpu.VMEM((CAP, C), jnp.float32)],
        compiler_params=pltpu.CompilerParams(
            dimension_semantics=("arbitrary", "arbitrary")),
    )(tok3, xb, W1, b1, W2, b2, wc)


def _combine_body(yd_hbm, g0_hbm, g1_hbm,
                  out_hbm,
                  g0_v, g1_v, bufa_v, bufb_v, bufc_v, gsem, wsem):
    c = lax.axis_index("c")
    s = lax.axis_index("s")
    wid = s * NC + c
    tbase = wid * TPT
    half = TPT // 2

    pltpu.sync_copy(g0_hbm.at[pl.ds(tbase, TPT)], g0_v)
    pltpu.sync_copy(g1_hbm.at[pl.ds(tbase, TPT)], g1_v)

    ga = pltpu.async_copy(yd_hbm.at[g0_v.at[pl.ds(0, half)]], bufa_v, gsem)
    gb = pltpu.async_copy(yd_hbm.at[g1_v.at[pl.ds(0, half)]], bufb_v, gsem)
    ga.wait()
    gb.wait()

    @plsc.parallel_loop(0, half * C // 16, 1, unroll=8)
    def _add0(i):
        r = i >> 6
        d = pl.ds((i & 63) * 16, 16)
        bufa_v[r, d] = bufa_v[r, d] + bufb_v[r, d]

    gc2 = pltpu.async_copy(yd_hbm.at[g0_v.at[pl.ds(half, half)]], bufc_v, gsem)
    gb2 = pltpu.async_copy(yd_hbm.at[g1_v.at[pl.ds(half, half)]], bufb_v, gsem)
    wa = pltpu.async_copy(bufa_v, out_hbm.at[pl.ds(tbase, half)], wsem)
    gc2.wait()
    gb2.wait()

    @plsc.parallel_loop(0, half * C // 16, 1, unroll=8)
    def _add1(i):
        r = i >> 6
        d = pl.ds((i & 63) * 16, 16)
        bufc_v[r, d] = bufc_v[r, d] + bufb_v[r, d]

    wa.wait()
    pltpu.sync_copy(bufc_v, out_hbm.at[pl.ds(tbase + half, half)])


def _combine(yd, g0, g1):
    mesh = plsc.VectorSubcoreMesh(core_axis_name="c", subcore_axis_name="s")
    return pl.kernel(
        _combine_body,
        out_type=jax.ShapeDtypeStruct((N, C), jnp.float32),
        mesh=mesh,
        compiler_params=pltpu.CompilerParams(needs_layout_passes=False),
        scratch_types=[
            pltpu.VMEM((TPT,), jnp.int32),
            pltpu.VMEM((TPT,), jnp.int32),
            pltpu.VMEM((TPT // 2, C), jnp.float32),
            pltpu.VMEM((TPT // 2, C), jnp.float32),
            pltpu.VMEM((TPT // 2, C), jnp.float32),
            pltpu.SemaphoreType.DMA,
            pltpu.SemaphoreType.DMA,
        ],
    )(yd, g0, g1)


def kernel(x, Wr, br, W1, b1, W2, b2):
    xf = x.reshape(N, C)
    lgs = xf @ Wr + br
    lg_p = jnp.pad(lgs, ((0, 0), (0, EP - E)), constant_values=-1e30)

    i1, i2, p1, p2 = _router(lg_p)
    tok, w, g0p, g1p, cnt = _scan(i1.reshape(N), i2.reshape(N),
                                  p1.reshape(N), p2.reshape(N))
    g0, g1 = _merge(g0p.reshape(E * N), g1p.reshape(E * N),
                    cnt.reshape(E * 16))
    yd = _ffn(tok.reshape(E, CAP, 1), xf.astype(jnp.bfloat16),
              W1, b1.reshape(E, 1, F), W2, b2.reshape(E, 1, C),
              w.reshape(E, CAP, 1))
    out = _combine(yd, g0, g1)
    return out.reshape(1, N, C)


# speedup vs baseline: 1.9899x; 1.0190x over previous
import jax
import jax.numpy as jnp
from jax import lax
from jax.experimental import pallas as pl
from jax.experimental.pallas import tpu as pltpu
from jax.experimental.pallas import tpu_sc as plsc

N = 2048
C = 1024
E = 8
F = 4096
CAP = 768
EP = 128
NS = 16
NC = 2
NW = NC * NS
ROWS = E * CAP
RPT = ROWS // NW
CHUNK = 64
NCHUNK = RPT // CHUNK
GC = 48
NGC = RPT // GC
TPT = N // NW
FB = 2048
NFB = F // FB


def _router_body(lg_ref, i1_ref, i2_ref, p1_ref, p2_ref):
    lg = lg_ref[...]
    iot = lax.broadcasted_iota(jnp.int32, (N, EP), 1)
    v1 = jnp.max(lg, axis=1, keepdims=True)
    i1 = jnp.min(jnp.where(lg >= v1, iot, EP), axis=1, keepdims=True)
    lg2 = jnp.where(iot == i1, -1e30, lg)
    v2 = jnp.max(lg2, axis=1, keepdims=True)
    i2 = jnp.min(jnp.where(lg2 >= v2, iot, EP), axis=1, keepdims=True)
    p1 = 1.0 / (1.0 + jnp.exp(v2 - v1))
    i1_ref[...] = i1
    i2_ref[...] = i2
    p1_ref[...] = p1
    p2_ref[...] = 1.0 - p1


def _router(lg_p):
    return pl.pallas_call(
        _router_body,
        out_shape=[
            jax.ShapeDtypeStruct((N, 1), jnp.int32),
            jax.ShapeDtypeStruct((N, 1), jnp.int32),
            jax.ShapeDtypeStruct((N, 1), jnp.float32),
            jax.ShapeDtypeStruct((N, 1), jnp.float32),
        ],
    )(lg_p)


def _scan_body(i1_hbm, i2_hbm, p1_hbm, p2_hbm,
               tok_hbm, w_hbm, g0p_hbm, g1p_hbm, cnt_hbm,
               i1_v, i2_v, p1_v, p2_v, tok_v, w_v, g0_v, g1_v, cnt_v):
    c = lax.axis_index("c")
    s = lax.axis_index("s")
    wid = s * NC + c

    @pl.when(wid < E)
    def _():
        e = wid
        pltpu.sync_copy(i1_hbm, i1_v)
        pltpu.sync_copy(i2_hbm, i2_v)
        pltpu.sync_copy(p1_hbm, p1_v)
        pltpu.sync_copy(p2_hbm, p2_v)

        z16i = jnp.zeros((16,), jnp.int32)
        z16f = jnp.zeros((16,), jnp.float32)

        def _zero_cap(i, _):
            tok_v[pl.ds(i * 16, 16)] = z16i
            w_v[pl.ds(i * 16, 16)] = z16f
            return 0
        lax.fori_loop(0, CAP // 16, _zero_cap, 0)

        def _zero_g(i, _):
            g0_v[pl.ds(i * 16, 16)] = z16i
            g1_v[pl.ds(i * 16, 16)] = z16i
            return 0
        lax.fori_loop(0, N // 16, _zero_g, 0)

        evec = z16i + e
        ebase = e * CAP

        def _scan(j, carry):
            a1 = i1_v[pl.ds(j * 16, 16)]
            a2 = i2_v[pl.ds(j * 16, 16)]
            m0 = a1 == evec
            m1 = a2 == evec
            o0 = jnp.where(m0, 1, 0).astype(jnp.int32)
            o1 = jnp.where(m1, 1, 0).astype(jnp.int32)
            c0 = plsc.cumsum(o0)
            c1 = plsc.cumsum(o1)
            prior = (c0 - o0) + (c1 - o1)
            r0 = carry + prior
            r1 = r0 + o0
            toks = lax.iota(jnp.int32, 16) + j * 16
            k0 = m0 & (r0 < CAP)
            k1 = m1 & (r1 < CAP)
            plsc.store_scatter(tok_v, [r0], toks, mask=k0)
            plsc.store_scatter(tok_v, [r1], toks, mask=k1)
            plsc.store_scatter(w_v, [r0], p1_v[pl.ds(j * 16, 16)], mask=k0)
            plsc.store_scatter(w_v, [r1], p2_v[pl.ds(j * 16, 16)], mask=k1)
            enc0 = jnp.where(k0, ebase + r0 + 2, 1)
            enc1 = jnp.where(k1, ebase + r1 + 2, 1)
            plsc.store_scatter(g0_v, [toks], enc0, mask=m0)
            plsc.store_scatter(g1_v, [toks], enc1, mask=m1)
            return (carry
                    + plsc.all_reduce_population_count(m0)
                    + plsc.all_reduce_population_count(m1))

        total = lax.fori_loop(0, N // 16, _scan, jnp.zeros((16,), jnp.int32))
        cnt_v[...] = total

        pltpu.sync_copy(tok_v, tok_hbm.at[e])
        pltpu.sync_copy(w_v, w_hbm.at[e])
        pltpu.sync_copy(g0_v, g0p_hbm.at[e])
        pltpu.sync_copy(g1_v, g1p_hbm.at[e])
        pltpu.sync_copy(cnt_v, cnt_hbm.at[e])


def _scan(i1f, i2f, p1f, p2f):
    mesh = plsc.VectorSubcoreMesh(core_axis_name="c", subcore_axis_name="s")
    return pl.kernel(
        _scan_body,
        out_type=[
            jax.ShapeDtypeStruct((E, CAP), jnp.int32),
            jax.ShapeDtypeStruct((E, CAP), jnp.float32),
            jax.ShapeDtypeStruct((E, N), jnp.int32),
            jax.ShapeDtypeStruct((E, N), jnp.int32),
            jax.ShapeDtypeStruct((E, 16), jnp.int32),
        ],
        mesh=mesh,
        compiler_params=pltpu.CompilerParams(needs_layout_passes=False),
        scratch_types=[
            pltpu.VMEM((N,), jnp.int32),
            pltpu.VMEM((N,), jnp.int32),
            pltpu.VMEM((N,), jnp.float32),
            pltpu.VMEM((N,), jnp.float32),
            pltpu.VMEM((CAP,), jnp.int32),
            pltpu.VMEM((CAP,), jnp.float32),
            pltpu.VMEM((N,), jnp.int32),
            pltpu.VMEM((N,), jnp.int32),
            pltpu.VMEM((16,), jnp.int32),
        ],
    )(i1f, i2f, p1f, p2f)


def _merge_body(g0pf_hbm, g1pf_hbm, cntf_hbm,
                g0_hbm, g1_hbm,
                gbuf_v, cnt_v, enc0_v, enc1_v):
    c = lax.axis_index("c")
    s = lax.axis_index("s")
    wid = s * NC + c
    tbase = wid * TPT

    pltpu.sync_copy(g0pf_hbm, gbuf_v)
    for i in range(TPT // 16):
        d = pl.ds(i * 16, 16)
        acc = gbuf_v[pl.ds(tbase + i * 16, 16)]
        for e in range(1, E):
            acc = acc + gbuf_v[pl.ds(e * N + tbase + i * 16, 16)]
        enc0_v[d] = acc
    pltpu.sync_copy(g1pf_hbm, gbuf_v)
    for i in range(TPT // 16):
        d = pl.ds(i * 16, 16)
        acc = gbuf_v[pl.ds(tbase + i * 16, 16)]
        for e in range(1, E):
            acc = acc + gbuf_v[pl.ds(e * N + tbase + i * 16, 16)]
        enc1_v[d] = acc

    pltpu.sync_copy(cntf_hbm, cnt_v)
    best = jnp.zeros((16,), jnp.int32) + (1 << 30)
    for e in range(E):
        best = jnp.minimum(best, cnt_v[pl.ds(e * 16, 16)] * 8 + e)
    zero_row = (best & 7) * CAP + (best >> 3)

    for i in range(TPT // 16):
        d = pl.ds(i * 16, 16)
        v0 = enc0_v[d]
        v1 = enc1_v[d]
        enc0_v[d] = jnp.where(v0 >= 2, v0 - 2, zero_row)
        enc1_v[d] = jnp.where(v1 >= 2, v1 - 2, zero_row)
    pltpu.sync_copy(enc0_v, g0_hbm.at[pl.ds(tbase, TPT)])
    pltpu.sync_copy(enc1_v, g1_hbm.at[pl.ds(tbase, TPT)])


def _merge(g0pf, g1pf, cntf):
    mesh = plsc.VectorSubcoreMesh(core_axis_name="c", subcore_axis_name="s")
    return pl.kernel(
        _merge_body,
        out_type=[
            jax.ShapeDtypeStruct((N,), jnp.int32),
            jax.ShapeDtypeStruct((N,), jnp.int32),
        ],
        mesh=mesh,
        compiler_params=pltpu.CompilerParams(needs_layout_passes=False),
        scratch_types=[
            pltpu.VMEM((E * N,), jnp.int32),
            pltpu.VMEM((E * 16,), jnp.int32),
            pltpu.VMEM((TPT,), jnp.int32),
            pltpu.VMEM((TPT,), jnp.int32),
        ],
    )(g0pf, g1pf, cntf)


def _ffn_body(tok_ref, xb_ref, w1_ref, b1_ref, w2_ref, b2_ref, wc_ref,
              yd_ref, xg_ref, acc_ref):
    fb = pl.program_id(1)

    @pl.when(fb == 0)
    def _():
        iot = lax.broadcasted_iota(jnp.int32, (CAP, N), 1)
        p = (iot == tok_ref[0]).astype(jnp.bfloat16)
        xg_ref[...] = jnp.dot(p, xb_ref[...],
                              preferred_element_type=jnp.float32
                              ).astype(jnp.bfloat16)

    h = jnp.dot(xg_ref[...], w1_ref[0].astype(jnp.bfloat16),
                preferred_element_type=jnp.float32) + b1_ref[0]
    h = jnp.maximum(h, 0.0)
    part = jnp.dot(h.astype(jnp.bfloat16), w2_ref[0].astype(jnp.bfloat16),
                   preferred_element_type=jnp.float32)

    @pl.when(fb == 0)
    def _():
        acc_ref[...] = part

    @pl.when(fb > 0)
    def _():
        acc_ref[...] += part

    @pl.when(fb == NFB - 1)
    def _():
        yd_ref[...] = (acc_ref[...] + b2_ref[0]) * wc_ref[0]


def _ffn(tok3, xb, W1, b1, W2, b2, wc):
    return pl.pallas_call(
        _ffn_body,
        grid=(E, NFB),
        in_specs=[
            pl.BlockSpec((1, CAP, 1), lambda e, f: (e, 0, 0)),
            pl.BlockSpec((N, C), lambda e, f: (0, 0)),
            pl.BlockSpec((1, C, FB), lambda e, f: (e, 0, f)),
            pl.BlockSpec((1, 1, FB), lambda e, f: (e, 0, f)),
            pl.BlockSpec((1, FB, C), lambda e, f: (e, f, 0)),
            pl.BlockSpec((1, 1, C), lambda e, f: (e, 0, 0)),
            pl.BlockSpec((1, CAP, 1), lambda e, f: (e, 0, 0)),
        ],
        out_specs=pl.BlockSpec((CAP, C), lambda e, f: (e, 0)),
        out_shape=jax.ShapeDtypeStruct((ROWS, C), jnp.float32),
        scratch_shapes=[pltpu.VMEM((CAP, C), jnp.bfloat16),
                        pltpu.VMEM((CAP, C), jnp.float32)],
        compiler_params=pltpu.CompilerParams(
            dimension_semantics=("arbitrary", "arbitrary")),
    )(tok3, xb, W1, b1, W2, b2, wc)


def _combine_body(yd_hbm, g0_hbm, g1_hbm,
                  out_hbm,
                  g0_v, g1_v, bufa_v, bufb_v, bufc_v, gsem, wsem):
    c = lax.axis_index("c")
    s = lax.axis_index("s")
    wid = s * NC + c
    tbase = wid * TPT
    half = TPT // 2

    pltpu.sync_copy(g0_hbm.at[pl.ds(tbase, TPT)], g0_v)
    pltpu.sync_copy(g1_hbm.at[pl.ds(tbase, TPT)], g1_v)

    ga = pltpu.async_copy(yd_hbm.at[g0_v.at[pl.ds(0, half)]], bufa_v, gsem)
    gb = pltpu.async_copy(yd_hbm.at[g1_v.at[pl.ds(0, half)]], bufb_v, gsem)
    ga.wait()
    gb.wait()

    @plsc.parallel_loop(0, half * C // 16, 1, unroll=8)
    def _add0(i):
        r = i >> 6
        d = pl.ds((i & 63) * 16, 16)
        bufa_v[r, d] = bufa_v[r, d] + bufb_v[r, d]

    gc2 = pltpu.async_copy(yd_hbm.at[g0_v.at[pl.ds(half, half)]], bufc_v, gsem)
    gb2 = pltpu.async_copy(yd_hbm.at[g1_v.at[pl.ds(half, half)]], bufb_v, gsem)
    wa = pltpu.async_copy(bufa_v, out_hbm.at[pl.ds(tbase, half)], wsem)
    gc2.wait()
    gb2.wait()

    @plsc.parallel_loop(0, half * C // 16, 1, unroll=8)
    def _add1(i):
        r = i >> 6
        d = pl.ds((i & 63) * 16, 16)
        bufc_v[r, d] = bufc_v[r, d] + bufb_v[r, d]

    wa.wait()
    pltpu.sync_copy(bufc_v, out_hbm.at[pl.ds(tbase + half, half)])


def _combine(yd, g0, g1):
    mesh = plsc.VectorSubcoreMesh(core_axis_name="c", subcore_axis_name="s")
    return pl.kernel(
        _combine_body,
        out_type=jax.ShapeDtypeStruct((N, C), jnp.float32),
        mesh=mesh,
        compiler_params=pltpu.CompilerParams(needs_layout_passes=False),
        scratch_types=[
            pltpu.VMEM((TPT,), jnp.int32),
            pltpu.VMEM((TPT,), jnp.int32),
            pltpu.VMEM((TPT // 2, C), jnp.float32),
            pltpu.VMEM((TPT // 2, C), jnp.float32),
            pltpu.VMEM((TPT // 2, C), jnp.float32),
            pltpu.SemaphoreType.DMA,
            pltpu.SemaphoreType.DMA,
        ],
    )(yd, g0, g1)


def kernel(x, Wr, br, W1, b1, W2, b2):
    xf = x.reshape(N, C)
    lgs = xf @ Wr + br
    lg_p = jnp.pad(lgs, ((0, 0), (0, EP - E)), constant_values=-1e30)

    i1, i2, p1, p2 = _router(lg_p)
    tok, w, g0p, g1p, cnt = _scan(i1.reshape(N), i2.reshape(N),
                                  p1.reshape(N), p2.reshape(N))
    g0, g1 = _merge(g0p.reshape(E * N), g1p.reshape(E * N),
                    cnt.reshape(E * 16))
    yd = _ffn(tok.reshape(E, CAP, 1), xf.astype(jnp.bfloat16),
              W1, b1.reshape(E, 1, F), W2, b2.reshape(E, 1, C),
              w.reshape(E, CAP, 1))
    out = _combine(yd, g0, g1)
    return out.reshape(1, N, C)


# speedup vs baseline: 2.1777x; 1.0944x over previous
import jax
import jax.numpy as jnp
from jax import lax
from jax.experimental import pallas as pl
from jax.experimental.pallas import tpu as pltpu
from jax.experimental.pallas import tpu_sc as plsc

N = 2048
C = 1024
E = 8
F = 4096
CAP = 768
EP = 128
NS = 16
NC = 2
NW = NC * NS
ROWS = E * CAP
RPT = ROWS // NW
CHUNK = 64
NCHUNK = RPT // CHUNK
GC = 48
NGC = RPT // GC
TPT = N // NW
TPT2 = N // NC // NS
FB = 2048
NFB = F // FB


def _scan_body(lgt_hbm,
               tok_hbm, w_hbm, g0_hbm, g1_hbm,
               lg_v, tok_v, w_v, g0_v, g1_v, cnt_v, gbuf_v, enc0_v, enc1_v,
               g0_s, g1_s, cnt_s):
    c = lax.axis_index("c")
    s = lax.axis_index("s")

    @pl.when(s < E)
    def _():
        e = s
        pltpu.sync_copy(lgt_hbm, lg_v)

        z16i = jnp.zeros((16,), jnp.int32)
        z16f = jnp.zeros((16,), jnp.float32)

        def _zero_cap(i, _):
            tok_v[pl.ds(i * 16, 16)] = z16i
            w_v[pl.ds(i * 16, 16)] = z16f
            return 0
        lax.fori_loop(0, CAP // 16, _zero_cap, 0)

        def _zero_g(i, _):
            g0_v[pl.ds(i * 16, 16)] = z16i
            g1_v[pl.ds(i * 16, 16)] = z16i
            return 0
        lax.fori_loop(0, N // 16, _zero_g, 0)

        ebase = e * CAP

        def _scan(j, carry):
            le = [lg_v[pl.ds(k * N + j * 16, 16)] for k in range(E)]
            v1 = le[0]
            for k in range(1, E):
                v1 = jnp.maximum(v1, le[k])
            i1 = z16i + (E - 1)
            for k in range(E - 2, -1, -1):
                i1 = jnp.where(le[k] >= v1, k, i1)
            le2 = [jnp.where(i1 == k, -1e30, le[k]) for k in range(E)]
            v2 = le2[0]
            for k in range(1, E):
                v2 = jnp.maximum(v2, le2[k])
            i2 = z16i + (E - 1)
            for k in range(E - 2, -1, -1):
                i2 = jnp.where(le2[k] >= v2, k, i2)
            p1 = 1.0 / (1.0 + jnp.exp(v2 - v1))
            p2 = 1.0 - p1

            m0 = i1 == e
            m1 = i2 == e
            o0 = jnp.where(m0, 1, 0).astype(jnp.int32)
            o1 = jnp.where(m1, 1, 0).astype(jnp.int32)
            c0 = plsc.cumsum(o0)
            c1 = plsc.cumsum(o1)
            prior = (c0 - o0) + (c1 - o1)
            r0 = carry + prior
            r1 = r0 + o0
            toks = lax.iota(jnp.int32, 16) + j * 16
            k0 = m0 & (r0 < CAP)
            k1 = m1 & (r1 < CAP)
            plsc.store_scatter(tok_v, [r0], toks, mask=k0)
            plsc.store_scatter(tok_v, [r1], toks, mask=k1)
            plsc.store_scatter(w_v, [r0], p1, mask=k0)
            plsc.store_scatter(w_v, [r1], p2, mask=k1)
            enc0 = jnp.where(k0, ebase + r0 + 2, 1)
            enc1 = jnp.where(k1, ebase + r1 + 2, 1)
            plsc.store_scatter(g0_v, [toks], enc0, mask=m0)
            plsc.store_scatter(g1_v, [toks], enc1, mask=m1)
            return (carry
                    + plsc.all_reduce_population_count(m0)
                    + plsc.all_reduce_population_count(m1))

        total = lax.fori_loop(0, N // 16, _scan, jnp.zeros((16,), jnp.int32))
        cnt_v[pl.ds(0, 16)] = total

        @pl.when(c == 0)
        def _():
            pltpu.sync_copy(tok_v, tok_hbm.at[e])
            pltpu.sync_copy(w_v, w_hbm.at[e])
        pltpu.sync_copy(g0_v, g0_s.at[pl.ds(e * N, N)])
        pltpu.sync_copy(g1_v, g1_s.at[pl.ds(e * N, N)])
        pltpu.sync_copy(cnt_v.at[pl.ds(0, 16)], cnt_s.at[pl.ds(e * 16, 16)])

    plsc.subcore_barrier()

    tbase = c * (N // NC) + s * TPT2
    for e in range(E):
        pltpu.sync_copy(g0_s.at[pl.ds(e * N + tbase, TPT2)],
                        gbuf_v.at[pl.ds(e * TPT2, TPT2)])
    pltpu.sync_copy(cnt_s, cnt_v)
    for i in range(TPT2 // 16):
        d = pl.ds(i * 16, 16)
        acc = gbuf_v[pl.ds(i * 16, 16)]
        for e in range(1, E):
            acc = acc + gbuf_v[pl.ds(e * TPT2 + i * 16, 16)]
        enc0_v[d] = acc
    for e in range(E):
        pltpu.sync_copy(g1_s.at[pl.ds(e * N + tbase, TPT2)],
                        gbuf_v.at[pl.ds(e * TPT2, TPT2)])
    for i in range(TPT2 // 16):
        d = pl.ds(i * 16, 16)
        acc = gbuf_v[pl.ds(i * 16, 16)]
        for e in range(1, E):
            acc = acc + gbuf_v[pl.ds(e * TPT2 + i * 16, 16)]
        enc1_v[d] = acc

    best = jnp.zeros((16,), jnp.int32) + (1 << 30)
    for e in range(E):
        best = jnp.minimum(best, cnt_v[pl.ds(e * 16, 16)] * 8 + e)
    zero_row = (best & 7) * CAP + (best >> 3)

    for i in range(TPT2 // 16):
        d = pl.ds(i * 16, 16)
        v0 = enc0_v[d]
        v1 = enc1_v[d]
        enc0_v[d] = jnp.where(v0 >= 2, v0 - 2, zero_row)
        enc1_v[d] = jnp.where(v1 >= 2, v1 - 2, zero_row)
    pltpu.sync_copy(enc0_v, g0_hbm.at[pl.ds(tbase, TPT2)])
    pltpu.sync_copy(enc1_v, g1_hbm.at[pl.ds(tbase, TPT2)])


def _scan(lgt):
    mesh = plsc.VectorSubcoreMesh(core_axis_name="c", subcore_axis_name="s")
    return pl.kernel(
        _scan_body,
        out_type=[
            jax.ShapeDtypeStruct((E, CAP), jnp.int32),
            jax.ShapeDtypeStruct((E, CAP), jnp.float32),
            jax.ShapeDtypeStruct((N,), jnp.int32),
            jax.ShapeDtypeStruct((N,), jnp.int32),
        ],
        mesh=mesh,
        compiler_params=pltpu.CompilerParams(needs_layout_passes=False),
        scratch_types=[
            pltpu.VMEM((E * N,), jnp.float32),
            pltpu.VMEM((CAP,), jnp.int32),
            pltpu.VMEM((CAP,), jnp.float32),
            pltpu.VMEM((N,), jnp.int32),
            pltpu.VMEM((N,), jnp.int32),
            pltpu.VMEM((E * 16,), jnp.int32),
            pltpu.VMEM((E * TPT2,), jnp.int32),
            pltpu.VMEM((TPT2,), jnp.int32),
            pltpu.VMEM((TPT2,), jnp.int32),
            pltpu.VMEM_SHARED((E * N,), jnp.int32),
            pltpu.VMEM_SHARED((E * N,), jnp.int32),
            pltpu.VMEM_SHARED((E * 16,), jnp.int32),
        ],
    )(lgt)


def _ffn_body(tok_ref, xb_ref, w1_ref, b1_ref, w2_ref, b2_ref, wc_ref,
              yd_ref, xg_ref, acc_ref):
    fb = pl.program_id(1)

    @pl.when(fb == 0)
    def _():
        iot = lax.broadcasted_iota(jnp.int32, (CAP, N), 1)
        p = (iot == tok_ref[0]).astype(jnp.bfloat16)
        xg_ref[...] = jnp.dot(p, xb_ref[...],
                              preferred_element_type=jnp.float32
                              ).astype(jnp.bfloat16)

    h = jnp.dot(xg_ref[...], w1_ref[0].astype(jnp.bfloat16),
                preferred_element_type=jnp.float32) + b1_ref[0]
    h = jnp.maximum(h, 0.0)
    part = jnp.dot(h.astype(jnp.bfloat16), w2_ref[0].astype(jnp.bfloat16),
                   preferred_element_type=jnp.float32)

    @pl.when(fb == 0)
    def _():
        acc_ref[...] = part

    @pl.when(fb > 0)
    def _():
        acc_ref[...] += part

    @pl.when(fb == NFB - 1)
    def _():
        yd_ref[...] = (acc_ref[...] + b2_ref[0]) * wc_ref[0]


def _ffn(tok3, xb, W1, b1, W2, b2, wc):
    return pl.pallas_call(
        _ffn_body,
        grid=(E, NFB),
        in_specs=[
            pl.BlockSpec((1, CAP, 1), lambda e, f: (e, 0, 0)),
            pl.BlockSpec((N, C), lambda e, f: (0, 0)),
            pl.BlockSpec((1, C, FB), lambda e, f: (e, 0, f)),
            pl.BlockSpec((1, 1, FB), lambda e, f: (e, 0, f)),
            pl.BlockSpec((1, FB, C), lambda e, f: (e, f, 0)),
            pl.BlockSpec((1, 1, C), lambda e, f: (e, 0, 0)),
            pl.BlockSpec((1, CAP, 1), lambda e, f: (e, 0, 0)),
        ],
        out_specs=pl.BlockSpec((CAP, C), lambda e, f: (e, 0)),
        out_shape=jax.ShapeDtypeStruct((ROWS, C), jnp.float32),
        scratch_shapes=[pltpu.VMEM((CAP, C), jnp.bfloat16),
                        pltpu.VMEM((CAP, C), jnp.float32)],
        compiler_params=pltpu.CompilerParams(
            dimension_semantics=("arbitrary", "arbitrary")),
    )(tok3, xb, W1, b1, W2, b2, wc)


def _combine_body(yd_hbm, g0_hbm, g1_hbm,
                  out_hbm,
                  g0_v, g1_v, bufa_v, bufb_v, bufc_v, gsem, wsem):
    c = lax.axis_index("c")
    s = lax.axis_index("s")
    wid = s * NC + c
    tbase = wid * TPT
    half = TPT // 2

    pltpu.sync_copy(g0_hbm.at[pl.ds(tbase, TPT)], g0_v)
    pltpu.sync_copy(g1_hbm.at[pl.ds(tbase, TPT)], g1_v)

    ga = pltpu.async_copy(yd_hbm.at[g0_v.at[pl.ds(0, half)]], bufa_v, gsem)
    gb = pltpu.async_copy(yd_hbm.at[g1_v.at[pl.ds(0, half)]], bufb_v, gsem)
    ga.wait()
    gb.wait()

    @plsc.parallel_loop(0, half * C // 16, 1, unroll=8)
    def _add0(i):
        r = i >> 6
        d = pl.ds((i & 63) * 16, 16)
        bufa_v[r, d] = bufa_v[r, d] + bufb_v[r, d]

    gc2 = pltpu.async_copy(yd_hbm.at[g0_v.at[pl.ds(half, half)]], bufc_v, gsem)
    gb2 = pltpu.async_copy(yd_hbm.at[g1_v.at[pl.ds(half, half)]], bufb_v, gsem)
    wa = pltpu.async_copy(bufa_v, out_hbm.at[pl.ds(tbase, half)], wsem)
    gc2.wait()
    gb2.wait()

    @plsc.parallel_loop(0, half * C // 16, 1, unroll=8)
    def _add1(i):
        r = i >> 6
        d = pl.ds((i & 63) * 16, 16)
        bufc_v[r, d] = bufc_v[r, d] + bufb_v[r, d]

    wa.wait()
    pltpu.sync_copy(bufc_v, out_hbm.at[pl.ds(tbase + half, half)])


def _combine(yd, g0, g1):
    mesh = plsc.VectorSubcoreMesh(core_axis_name="c", subcore_axis_name="s")
    return pl.kernel(
        _combine_body,
        out_type=jax.ShapeDtypeStruct((N, C), jnp.float32),
        mesh=mesh,
        compiler_params=pltpu.CompilerParams(needs_layout_passes=False),
        scratch_types=[
            pltpu.VMEM((TPT,), jnp.int32),
            pltpu.VMEM((TPT,), jnp.int32),
            pltpu.VMEM((TPT // 2, C), jnp.float32),
            pltpu.VMEM((TPT // 2, C), jnp.float32),
            pltpu.VMEM((TPT // 2, C), jnp.float32),
            pltpu.SemaphoreType.DMA,
            pltpu.SemaphoreType.DMA,
        ],
    )(yd, g0, g1)


def kernel(x, Wr, br, W1, b1, W2, b2):
    xf = x.reshape(N, C)
    lgs = xf @ Wr + br
    lgt = lgs.T.reshape(E * N)

    tok, w, g0, g1 = _scan(lgt)
    yd = _ffn(tok.reshape(E, CAP, 1), xf.astype(jnp.bfloat16),
              W1, b1.reshape(E, 1, F), W2, b2.reshape(E, 1, C),
              w.reshape(E, CAP, 1))
    out = _combine(yd, g0, g1)
    return out.reshape(1, N, C)


# speedup vs baseline: 2.1818x; 1.0018x over previous
import jax
import jax.numpy as jnp
from jax import lax
from jax.experimental import pallas as pl
from jax.experimental.pallas import tpu as pltpu
from jax.experimental.pallas import tpu_sc as plsc

N = 2048
C = 1024
E = 8
F = 4096
CAP = 768
EP = 128
NS = 16
NC = 2
NW = NC * NS
ROWS = E * CAP
RPT = ROWS // NW
CHUNK = 64
NCHUNK = RPT // CHUNK
GC = 48
NGC = RPT // GC
TPT = N // NW
TPT2 = N // NC // NS
FB = 2048
NFB = F // FB


def _scan_body(lgt_hbm,
               tok_hbm, w_hbm, g0_hbm, g1_hbm,
               lg_v, tok_v, w_v, g0_v, g1_v, cnt_v, gbuf_v, enc0_v, enc1_v,
               g0_s, g1_s, cnt_s):
    c = lax.axis_index("c")
    s = lax.axis_index("s")

    @pl.when(s < E)
    def _():
        e = s
        pltpu.sync_copy(lgt_hbm, lg_v)

        z16i = jnp.zeros((16,), jnp.int32)
        z16f = jnp.zeros((16,), jnp.float32)

        def _zero_cap(i, _):
            tok_v[pl.ds(i * 16, 16)] = z16i
            w_v[pl.ds(i * 16, 16)] = z16f
            return 0
        lax.fori_loop(0, CAP // 16, _zero_cap, 0)

        def _zero_g(i, _):
            g0_v[pl.ds(i * 16, 16)] = z16i
            g1_v[pl.ds(i * 16, 16)] = z16i
            return 0
        lax.fori_loop(0, N // 16, _zero_g, 0)

        ebase = e * CAP

        def _scan(j, carry):
            le = [lg_v[pl.ds(k * N + j * 16, 16)] for k in range(E)]
            v1 = le[0]
            for k in range(1, E):
                v1 = jnp.maximum(v1, le[k])
            i1 = z16i + (E - 1)
            for k in range(E - 2, -1, -1):
                i1 = jnp.where(le[k] >= v1, k, i1)
            le2 = [jnp.where(i1 == k, -1e30, le[k]) for k in range(E)]
            v2 = le2[0]
            for k in range(1, E):
                v2 = jnp.maximum(v2, le2[k])
            i2 = z16i + (E - 1)
            for k in range(E - 2, -1, -1):
                i2 = jnp.where(le2[k] >= v2, k, i2)
            p1 = 1.0 / (1.0 + jnp.exp(v2 - v1))
            p2 = 1.0 - p1

            m0 = i1 == e
            m1 = i2 == e
            o0 = jnp.where(m0, 1, 0).astype(jnp.int32)
            o1 = jnp.where(m1, 1, 0).astype(jnp.int32)
            c0 = plsc.cumsum(o0)
            c1 = plsc.cumsum(o1)
            prior = (c0 - o0) + (c1 - o1)
            r0 = carry + prior
            r1 = r0 + o0
            toks = lax.iota(jnp.int32, 16) + j * 16
            k0 = m0 & (r0 < CAP)
            k1 = m1 & (r1 < CAP)
            plsc.store_scatter(tok_v, [r0], toks, mask=k0)
            plsc.store_scatter(tok_v, [r1], toks, mask=k1)
            plsc.store_scatter(w_v, [r0], p1, mask=k0)
            plsc.store_scatter(w_v, [r1], p2, mask=k1)
            enc0 = jnp.where(k0, ebase + r0 + 2, 1)
            enc1 = jnp.where(k1, ebase + r1 + 2, 1)
            plsc.store_scatter(g0_v, [toks], enc0, mask=m0)
            plsc.store_scatter(g1_v, [toks], enc1, mask=m1)
            return (carry
                    + plsc.all_reduce_population_count(m0)
                    + plsc.all_reduce_population_count(m1))

        total = lax.fori_loop(0, N // 16, _scan, jnp.zeros((16,), jnp.int32))
        cnt_v[pl.ds(0, 16)] = total

        @pl.when(c == 0)
        def _():
            pltpu.sync_copy(tok_v, tok_hbm.at[e])
            pltpu.sync_copy(w_v, w_hbm.at[e])
        pltpu.sync_copy(g0_v, g0_s.at[pl.ds(e * N, N)])
        pltpu.sync_copy(g1_v, g1_s.at[pl.ds(e * N, N)])
        pltpu.sync_copy(cnt_v.at[pl.ds(0, 16)], cnt_s.at[pl.ds(e * 16, 16)])

    plsc.subcore_barrier()

    tbase = c * (N // NC) + s * TPT2
    for e in range(E):
        pltpu.sync_copy(g0_s.at[pl.ds(e * N + tbase, TPT2)],
                        gbuf_v.at[pl.ds(e * TPT2, TPT2)])
    pltpu.sync_copy(cnt_s, cnt_v)
    for i in range(TPT2 // 16):
        d = pl.ds(i * 16, 16)
        acc = gbuf_v[pl.ds(i * 16, 16)]
        for e in range(1, E):
            acc = acc + gbuf_v[pl.ds(e * TPT2 + i * 16, 16)]
        enc0_v[d] = acc
    for e in range(E):
        pltpu.sync_copy(g1_s.at[pl.ds(e * N + tbase, TPT2)],
                        gbuf_v.at[pl.ds(e * TPT2, TPT2)])
    for i in range(TPT2 // 16):
        d = pl.ds(i * 16, 16)
        acc = gbuf_v[pl.ds(i * 16, 16)]
        for e in range(1, E):
            acc = acc + gbuf_v[pl.ds(e * TPT2 + i * 16, 16)]
        enc1_v[d] = acc

    best = jnp.zeros((16,), jnp.int32) + (1 << 30)
    for e in range(E):
        best = jnp.minimum(best, cnt_v[pl.ds(e * 16, 16)] * 8 + e)
    zero_row = (best & 7) * CAP + (best >> 3)

    for i in range(TPT2 // 16):
        d = pl.ds(i * 16, 16)
        v0 = enc0_v[d]
        v1 = enc1_v[d]
        enc0_v[d] = jnp.where(v0 >= 2, v0 - 2, zero_row)
        enc1_v[d] = jnp.where(v1 >= 2, v1 - 2, zero_row)
    pltpu.sync_copy(enc0_v, g0_hbm.at[pl.ds(tbase, TPT2)])
    pltpu.sync_copy(enc1_v, g1_hbm.at[pl.ds(tbase, TPT2)])


def _scan(lgt):
    mesh = plsc.VectorSubcoreMesh(core_axis_name="c", subcore_axis_name="s")
    return pl.kernel(
        _scan_body,
        out_type=[
            jax.ShapeDtypeStruct((E, CAP), jnp.int32),
            jax.ShapeDtypeStruct((E, CAP), jnp.float32),
            jax.ShapeDtypeStruct((N,), jnp.int32),
            jax.ShapeDtypeStruct((N,), jnp.int32),
        ],
        mesh=mesh,
        compiler_params=pltpu.CompilerParams(needs_layout_passes=False),
        scratch_types=[
            pltpu.VMEM((E * N,), jnp.float32),
            pltpu.VMEM((CAP,), jnp.int32),
            pltpu.VMEM((CAP,), jnp.float32),
            pltpu.VMEM((N,), jnp.int32),
            pltpu.VMEM((N,), jnp.int32),
            pltpu.VMEM((E * 16,), jnp.int32),
            pltpu.VMEM((E * TPT2,), jnp.int32),
            pltpu.VMEM((TPT2,), jnp.int32),
            pltpu.VMEM((TPT2,), jnp.int32),
            pltpu.VMEM_SHARED((E * N,), jnp.int32),
            pltpu.VMEM_SHARED((E * N,), jnp.int32),
            pltpu.VMEM_SHARED((E * 16,), jnp.int32),
        ],
    )(lgt)


def _ffn_body(tok_ref, xb_ref, w1_ref, b1_ref, w2_ref, b2_ref, wc_ref,
              yd_ref, xg_ref, acc_ref):
    fb = pl.program_id(1)

    @pl.when(fb == 0)
    def _():
        iot = lax.broadcasted_iota(jnp.int32, (CAP, N), 1)
        p = (iot == tok_ref[0]).astype(jnp.bfloat16)
        xg_ref[...] = jnp.dot(p, xb_ref[...],
                              preferred_element_type=jnp.float32
                              ).astype(jnp.bfloat16)

    xg = xg_ref[...]
    hb = []
    for q in range(2):
        fsl = pl.ds(q * (FB // 2), FB // 2)
        a = jnp.dot(xg, w1_ref[0, :, fsl].astype(jnp.bfloat16),
                    preferred_element_type=jnp.float32) + b1_ref[0, :, fsl]
        hb.append(jnp.maximum(a, 0.0).astype(jnp.bfloat16))
    part = jnp.dot(hb[0], w2_ref[0, pl.ds(0, FB // 2), :].astype(jnp.bfloat16),
                   preferred_element_type=jnp.float32)
    part = part + jnp.dot(
        hb[1], w2_ref[0, pl.ds(FB // 2, FB // 2), :].astype(jnp.bfloat16),
        preferred_element_type=jnp.float32)

    @pl.when(fb == 0)
    def _():
        acc_ref[...] = part

    @pl.when(fb > 0)
    def _():
        acc_ref[...] += part

    @pl.when(fb == NFB - 1)
    def _():
        yd_ref[...] = (acc_ref[...] + b2_ref[0]) * wc_ref[0]


def _ffn(tok3, xb, W1, b1, W2, b2, wc):
    return pl.pallas_call(
        _ffn_body,
        grid=(E, NFB),
        in_specs=[
            pl.BlockSpec((1, CAP, 1), lambda e, f: (e, 0, 0)),
            pl.BlockSpec((N, C), lambda e, f: (0, 0)),
            pl.BlockSpec((1, C, FB), lambda e, f: (e, 0, f)),
            pl.BlockSpec((1, 1, FB), lambda e, f: (e, 0, f)),
            pl.BlockSpec((1, FB, C), lambda e, f: (e, f, 0)),
            pl.BlockSpec((1, 1, C), lambda e, f: (e, 0, 0)),
            pl.BlockSpec((1, CAP, 1), lambda e, f: (e, 0, 0)),
        ],
        out_specs=pl.BlockSpec((CAP, C), lambda e, f: (e, 0)),
        out_shape=jax.ShapeDtypeStruct((ROWS, C), jnp.float32),
        scratch_shapes=[pltpu.VMEM((CAP, C), jnp.bfloat16),
                        pltpu.VMEM((CAP, C), jnp.float32)],
        compiler_params=pltpu.CompilerParams(
            dimension_semantics=("arbitrary", "arbitrary")),
    )(tok3, xb, W1, b1, W2, b2, wc)


def _combine_body(yd_hbm, g0_hbm, g1_hbm,
                  out_hbm,
                  g0_v, g1_v, bufa_v, bufb_v, bufc_v, gsem, wsem):
    c = lax.axis_index("c")
    s = lax.axis_index("s")
    wid = s * NC + c
    tbase = wid * TPT
    half = TPT // 2

    pltpu.sync_copy(g0_hbm.at[pl.ds(tbase, TPT)], g0_v)
    pltpu.sync_copy(g1_hbm.at[pl.ds(tbase, TPT)], g1_v)

    ga = pltpu.async_copy(yd_hbm.at[g0_v.at[pl.ds(0, half)]], bufa_v, gsem)
    gb = pltpu.async_copy(yd_hbm.at[g1_v.at[pl.ds(0, half)]], bufb_v, gsem)
    ga.wait()
    gb.wait()

    @plsc.parallel_loop(0, half * C // 16, 1, unroll=8)
    def _add0(i):
        r = i >> 6
        d = pl.ds((i & 63) * 16, 16)
        bufa_v[r, d] = bufa_v[r, d] + bufb_v[r, d]

    gc2 = pltpu.async_copy(yd_hbm.at[g0_v.at[pl.ds(half, half)]], bufc_v, gsem)
    gb2 = pltpu.async_copy(yd_hbm.at[g1_v.at[pl.ds(half, half)]], bufb_v, gsem)
    wa = pltpu.async_copy(bufa_v, out_hbm.at[pl.ds(tbase, half)], wsem)
    gc2.wait()
    gb2.wait()

    @plsc.parallel_loop(0, half * C // 16, 1, unroll=8)
    def _add1(i):
        r = i >> 6
        d = pl.ds((i & 63) * 16, 16)
        bufc_v[r, d] = bufc_v[r, d] + bufb_v[r, d]

    wa.wait()
    pltpu.sync_copy(bufc_v, out_hbm.at[pl.ds(tbase + half, half)])


def _combine(yd, g0, g1):
    mesh = plsc.VectorSubcoreMesh(core_axis_name="c", subcore_axis_name="s")
    return pl.kernel(
        _combine_body,
        out_type=jax.ShapeDtypeStruct((N, C), jnp.float32),
        mesh=mesh,
        compiler_params=pltpu.CompilerParams(needs_layout_passes=False),
        scratch_types=[
            pltpu.VMEM((TPT,), jnp.int32),
            pltpu.VMEM((TPT,), jnp.int32),
            pltpu.VMEM((TPT // 2, C), jnp.float32),
            pltpu.VMEM((TPT // 2, C), jnp.float32),
            pltpu.VMEM((TPT // 2, C), jnp.float32),
            pltpu.SemaphoreType.DMA,
            pltpu.SemaphoreType.DMA,
        ],
    )(yd, g0, g1)


def kernel(x, Wr, br, W1, b1, W2, b2):
    xf = x.reshape(N, C)
    lgs = xf @ Wr + br
    lgt = lgs.T.reshape(E * N)

    tok, w, g0, g1 = _scan(lgt)
    yd = _ffn(tok.reshape(E, CAP, 1), xf.astype(jnp.bfloat16),
              W1, b1.reshape(E, 1, F), W2, b2.reshape(E, 1, C),
              w.reshape(E, CAP, 1))
    out = _combine(yd, g0, g1)
    return out.reshape(1, N, C)


# speedup vs baseline: 2.1843x; 1.0011x over previous
import jax
import jax.numpy as jnp
from jax import lax
from jax.experimental import pallas as pl
from jax.experimental.pallas import tpu as pltpu
from jax.experimental.pallas import tpu_sc as plsc

N = 2048
C = 1024
E = 8
F = 4096
CAP = 768
EP = 128
NS = 16
NC = 2
NW = NC * NS
ROWS = E * CAP
RPT = ROWS // NW
CHUNK = 64
NCHUNK = RPT // CHUNK
GC = 48
NGC = RPT // GC
TPT = N // NW
TPT2 = N // NC // NS
FB = 2048
NFB = F // FB


def _scan_body(lgt_hbm,
               tok_hbm, w_hbm, g0_hbm, g1_hbm,
               lg_v, tok_v, w_v, g0_v, g1_v, cnt_v, gbuf_v, enc0_v, enc1_v,
               g0_s, g1_s, cnt_s):
    c = lax.axis_index("c")
    s = lax.axis_index("s")

    @pl.when(s < E)
    def _():
        e = s
        pltpu.sync_copy(lgt_hbm, lg_v)

        z16i = jnp.zeros((16,), jnp.int32)
        z16f = jnp.zeros((16,), jnp.float32)

        def _zero_cap(i, _):
            tok_v[pl.ds(i * 16, 16)] = z16i
            w_v[pl.ds(i * 16, 16)] = z16f
            return 0
        lax.fori_loop(0, CAP // 16, _zero_cap, 0)

        def _zero_g(i, _):
            g0_v[pl.ds(i * 16, 16)] = z16i
            g1_v[pl.ds(i * 16, 16)] = z16i
            return 0
        lax.fori_loop(0, N // 16, _zero_g, 0)

        ebase = e * CAP

        def _scan(j, carry):
            le = [lg_v[pl.ds(k * N + j * 16, 16)] for k in range(E)]
            v1 = le[0]
            for k in range(1, E):
                v1 = jnp.maximum(v1, le[k])
            i1 = z16i + (E - 1)
            for k in range(E - 2, -1, -1):
                i1 = jnp.where(le[k] >= v1, k, i1)
            le2 = [jnp.where(i1 == k, -1e30, le[k]) for k in range(E)]
            v2 = le2[0]
            for k in range(1, E):
                v2 = jnp.maximum(v2, le2[k])
            i2 = z16i + (E - 1)
            for k in range(E - 2, -1, -1):
                i2 = jnp.where(le2[k] >= v2, k, i2)
            p1 = 1.0 / (1.0 + jnp.exp(v2 - v1))
            p2 = 1.0 - p1

            m0 = i1 == e
            m1 = i2 == e
            o0 = jnp.where(m0, 1, 0).astype(jnp.int32)
            o1 = jnp.where(m1, 1, 0).astype(jnp.int32)
            c0 = plsc.cumsum(o0)
            c1 = plsc.cumsum(o1)
            prior = (c0 - o0) + (c1 - o1)
            r0 = carry + prior
            r1 = r0 + o0
            toks = lax.iota(jnp.int32, 16) + j * 16
            k0 = m0 & (r0 < CAP)
            k1 = m1 & (r1 < CAP)
            plsc.store_scatter(tok_v, [r0], toks, mask=k0)
            plsc.store_scatter(tok_v, [r1], toks, mask=k1)
            plsc.store_scatter(w_v, [r0], p1, mask=k0)
            plsc.store_scatter(w_v, [r1], p2, mask=k1)
            enc0 = jnp.where(k0, ebase + r0 + 2, 1)
            enc1 = jnp.where(k1, ebase + r1 + 2, 1)
            plsc.store_scatter(g0_v, [toks], enc0, mask=m0)
            plsc.store_scatter(g1_v, [toks], enc1, mask=m1)
            return (carry
                    + plsc.all_reduce_population_count(m0)
                    + plsc.all_reduce_population_count(m1))

        total = lax.fori_loop(0, N // 16, _scan, jnp.zeros((16,), jnp.int32))
        cnt_v[pl.ds(0, 16)] = total

        @pl.when(c == 0)
        def _():
            pltpu.sync_copy(tok_v, tok_hbm.at[e])
            pltpu.sync_copy(w_v, w_hbm.at[e])
        pltpu.sync_copy(g0_v, g0_s.at[pl.ds(e * N, N)])
        pltpu.sync_copy(g1_v, g1_s.at[pl.ds(e * N, N)])
        pltpu.sync_copy(cnt_v.at[pl.ds(0, 16)], cnt_s.at[pl.ds(e * 16, 16)])

    plsc.subcore_barrier()

    tbase = c * (N // NC) + s * TPT2
    for e in range(E):
        pltpu.sync_copy(g0_s.at[pl.ds(e * N + tbase, TPT2)],
                        gbuf_v.at[pl.ds(e * TPT2, TPT2)])
    pltpu.sync_copy(cnt_s, cnt_v)
    for i in range(TPT2 // 16):
        d = pl.ds(i * 16, 16)
        acc = gbuf_v[pl.ds(i * 16, 16)]
        for e in range(1, E):
            acc = acc + gbuf_v[pl.ds(e * TPT2 + i * 16, 16)]
        enc0_v[d] = acc
    for e in range(E):
        pltpu.sync_copy(g1_s.at[pl.ds(e * N + tbase, TPT2)],
                        gbuf_v.at[pl.ds(e * TPT2, TPT2)])
    for i in range(TPT2 // 16):
        d = pl.ds(i * 16, 16)
        acc = gbuf_v[pl.ds(i * 16, 16)]
        for e in range(1, E):
            acc = acc + gbuf_v[pl.ds(e * TPT2 + i * 16, 16)]
        enc1_v[d] = acc

    best = jnp.zeros((16,), jnp.int32) + (1 << 30)
    for e in range(E):
        best = jnp.minimum(best, cnt_v[pl.ds(e * 16, 16)] * 8 + e)
    zero_row = (best & 7) * CAP + (best >> 3)

    for i in range(TPT2 // 16):
        d = pl.ds(i * 16, 16)
        v0 = enc0_v[d]
        v1 = enc1_v[d]
        enc0_v[d] = jnp.where(v0 >= 2, v0 - 2, zero_row)
        enc1_v[d] = jnp.where(v1 >= 2, v1 - 2, zero_row)
    pltpu.sync_copy(enc0_v, g0_hbm.at[pl.ds(tbase, TPT2)])
    pltpu.sync_copy(enc1_v, g1_hbm.at[pl.ds(tbase, TPT2)])


def _scan(lgt):
    mesh = plsc.VectorSubcoreMesh(core_axis_name="c", subcore_axis_name="s")
    return pl.kernel(
        _scan_body,
        out_type=[
            jax.ShapeDtypeStruct((E, CAP), jnp.int32),
            jax.ShapeDtypeStruct((E, CAP), jnp.float32),
            jax.ShapeDtypeStruct((N,), jnp.int32),
            jax.ShapeDtypeStruct((N,), jnp.int32),
        ],
        mesh=mesh,
        compiler_params=pltpu.CompilerParams(needs_layout_passes=False),
        scratch_types=[
            pltpu.VMEM((E * N,), jnp.float32),
            pltpu.VMEM((CAP,), jnp.int32),
            pltpu.VMEM((CAP,), jnp.float32),
            pltpu.VMEM((N,), jnp.int32),
            pltpu.VMEM((N,), jnp.int32),
            pltpu.VMEM((E * 16,), jnp.int32),
            pltpu.VMEM((E * TPT2,), jnp.int32),
            pltpu.VMEM((TPT2,), jnp.int32),
            pltpu.VMEM((TPT2,), jnp.int32),
            pltpu.VMEM_SHARED((E * N,), jnp.int32),
            pltpu.VMEM_SHARED((E * N,), jnp.int32),
            pltpu.VMEM_SHARED((E * 16,), jnp.int32),
        ],
    )(lgt)


def _ffn_body(tok_ref, xb_ref, w1_ref, b1_ref, w2_ref, b2_ref, wc_ref,
              yd_ref, xg_ref, acc_ref):
    fb = pl.program_id(1)

    @pl.when(fb == 0)
    def _():
        iot = lax.broadcasted_iota(jnp.int32, (CAP, N), 1)
        p = (iot == tok_ref[0]).astype(jnp.bfloat16)
        xg_ref[...] = jnp.dot(p, xb_ref[...],
                              preferred_element_type=jnp.float32
                              ).astype(jnp.bfloat16)

    h = jnp.dot(xg_ref[...], w1_ref[0].astype(jnp.bfloat16),
                preferred_element_type=jnp.float32) + b1_ref[0]
    h = jnp.maximum(h, 0.0)
    part = jnp.dot(h.astype(jnp.bfloat16), w2_ref[0].astype(jnp.bfloat16),
                   preferred_element_type=jnp.float32)

    @pl.when(fb == 0)
    def _():
        acc_ref[...] = part

    @pl.when(fb > 0)
    def _():
        acc_ref[...] += part

    @pl.when(fb == NFB - 1)
    def _():
        yd_ref[...] = (acc_ref[...] + b2_ref[0]) * wc_ref[0]


def _ffn(tok3, xb, W1, b1, W2, b2, wc):
    return pl.pallas_call(
        _ffn_body,
        grid=(E, NFB),
        in_specs=[
            pl.BlockSpec((1, CAP, 1), lambda e, f: (e, 0, 0)),
            pl.BlockSpec((N, C), lambda e, f: (0, 0)),
            pl.BlockSpec((1, C, FB), lambda e, f: (e, 0, f)),
            pl.BlockSpec((1, 1, FB), lambda e, f: (e, 0, f)),
            pl.BlockSpec((1, FB, C), lambda e, f: (e, f, 0)),
            pl.BlockSpec((1, 1, C), lambda e, f: (e, 0, 0)),
            pl.BlockSpec((1, CAP, 1), lambda e, f: (e, 0, 0)),
        ],
        out_specs=pl.BlockSpec((CAP, C), lambda e, f: (e, 0)),
        out_shape=jax.ShapeDtypeStruct((ROWS, C), jnp.float32),
        scratch_shapes=[pltpu.VMEM((CAP, C), jnp.bfloat16),
                        pltpu.VMEM((CAP, C), jnp.float32)],
        compiler_params=pltpu.CompilerParams(
            dimension_semantics=("arbitrary", "arbitrary")),
    )(tok3, xb, W1, b1, W2, b2, wc)


def _combine_body(yd_hbm, g0_hbm, g1_hbm,
                  out_hbm,
                  g0_v, g1_v, bufa_v, bufb_v, bufc_v, gsem, wsem):
    c = lax.axis_index("c")
    s = lax.axis_index("s")
    wid = s * NC + c
    tbase = wid * TPT
    half = TPT // 2

    pltpu.sync_copy(g0_hbm.at[pl.ds(tbase, TPT)], g0_v)
    pltpu.sync_copy(g1_hbm.at[pl.ds(tbase, TPT)], g1_v)

    ga = pltpu.async_copy(yd_hbm.at[g0_v.at[pl.ds(0, half)]], bufa_v, gsem)
    gb = pltpu.async_copy(yd_hbm.at[g1_v.at[pl.ds(0, half)]], bufb_v, gsem)
    ga.wait()
    gb.wait()

    @plsc.parallel_loop(0, half * C // 16, 1, unroll=8)
    def _add0(i):
        r = i >> 6
        d = pl.ds((i & 63) * 16, 16)
        bufa_v[r, d] = bufa_v[r, d] + bufb_v[r, d]

    gc2 = pltpu.async_copy(yd_hbm.at[g0_v.at[pl.ds(half, half)]], bufc_v, gsem)
    gb2 = pltpu.async_copy(yd_hbm.at[g1_v.at[pl.ds(half, half)]], bufb_v, gsem)
    wa = pltpu.async_copy(bufa_v, out_hbm.at[pl.ds(tbase, half)], wsem)
    gc2.wait()
    gb2.wait()

    @plsc.parallel_loop(0, half * C // 16, 1, unroll=8)
    def _add1(i):
        r = i >> 6
        d = pl.ds((i & 63) * 16, 16)
        bufc_v[r, d] = bufc_v[r, d] + bufb_v[r, d]

    wa.wait()
    pltpu.sync_copy(bufc_v, out_hbm.at[pl.ds(tbase + half, half)])


def _combine(yd, g0, g1):
    mesh = plsc.VectorSubcoreMesh(core_axis_name="c", subcore_axis_name="s")
    return pl.kernel(
        _combine_body,
        out_type=jax.ShapeDtypeStruct((N, C), jnp.float32),
        mesh=mesh,
        compiler_params=pltpu.CompilerParams(needs_layout_passes=False),
        scratch_types=[
            pltpu.VMEM((TPT,), jnp.int32),
            pltpu.VMEM((TPT,), jnp.int32),
            pltpu.VMEM((TPT // 2, C), jnp.float32),
            pltpu.VMEM((TPT // 2, C), jnp.float32),
            pltpu.VMEM((TPT // 2, C), jnp.float32),
            pltpu.SemaphoreType.DMA,
            pltpu.SemaphoreType.DMA,
        ],
    )(yd, g0, g1)


def kernel(x, Wr, br, W1, b1, W2, b2):
    xf = x.reshape(N, C)
    lgs = xf @ Wr + br
    lgt = lgs.T.reshape(E * N)

    tok, w, g0, g1 = _scan(lgt)
    yd = _ffn(tok.reshape(E, CAP, 1), xf.astype(jnp.bfloat16),
              W1, b1.reshape(E, 1, F), W2, b2.reshape(E, 1, C),
              w.reshape(E, CAP, 1))
    out = _combine(yd, g0, g1)
    return out.reshape(1, N, C)
